# baseline (speedup 1.0000x reference)
import jax
import jax.numpy as jnp
from jax import lax
from jax.experimental import pallas as pl
from jax.experimental.pallas import tpu as pltpu
from jax.experimental.pallas import tpu_sc as plsc

EMB = 64
SCALE = 8.0
NC = 2
NS = 16
NW = NC * NS
CHUNK = 128
LANES = 16
NBUF = 4


def _body(tokens_hbm, table_hbm, out_hbm, idx_v, gbuf, tbuf, *sems):
    sem_g = sems[:NBUF]
    sem_w = sems[NBUF:]
    wid = lax.axis_index("s") * NC + lax.axis_index("c")
    nch = tokens_hbm.shape[0]
    pltpu.sync_copy(tokens_hbm.at[:, pl.ds(wid * CHUNK, CHUNK)], idx_v)

    def gather(j, b):
        pltpu.async_copy(table_hbm.at[idx_v.at[j]], gbuf.at[b], sem_g[b])

    def wait_gather(b):
        pltpu.make_async_copy(
            table_hbm.at[idx_v.at[0]], gbuf.at[b], sem_g[b]).wait()

    def writeback(j, b):
        pltpu.async_copy(
            tbuf.at[b, slice(None), slice(None), pl.ds(0, CHUNK)],
            out_hbm.at[j, slice(None), wid], sem_w[b])

    def wait_writeback(b):
        pltpu.make_async_copy(
            tbuf.at[b, slice(None), slice(None), pl.ds(0, CHUNK)],
            out_hbm.at[0, slice(None), wid], sem_w[b]).wait()

    for b in range(NBUF):
        gather(b, b)

    lanes = lax.iota(jnp.int32, LANES)
    def _idx(g, parity):
        e = jnp.full((LANES,), g * 32 + parity, jnp.int32) + 2 * lanes
        return e >> 3, e & 7

    eidx = [[_idx(g, parity) for parity in (0, 1)] for g in range(EMB // 32)]

    def group(g, carry):
        for b in range(NBUF):
            j = g * NBUF + b
            wait_gather(b)

            @pl.when(j >= NBUF)
            def _():
                wait_writeback(b)

            @plsc.parallel_loop(0, CHUNK, unroll=2)
            def _pass(t):
                t16 = jnp.full((LANES,), 0, jnp.int32) + t
                for c in range(EMB // 32):
                    v32 = gbuf[b, t, pl.ds(c * 32, 32)]
                    va, vb = plsc.unpack(
                        v32, format=plsc.PackFormat.INTERLEAVED,
                        preferred_element_type=jnp.float32)
                    for parity, v in ((0, va), (1, vb)):
                        eb2, ei2 = eidx[c][parity]
                        plsc.store_scatter(
                            tbuf.at[b], [eb2, ei2, t16], v * SCALE)

            writeback(j, b)

            @pl.when(j + NBUF < nch)
            def _():
                gather(j + NBUF, b)
        return carry

    lax.fori_loop(0, nch // NBUF, group, None)

    for b in range(NBUF):
        wait_writeback(b)


def kernel(tokens, embedding_weight):
    B, L = tokens.shape
    nbb = B // CHUNK
    assert nbb == NW and L % NBUF == 0, (B, L)
    tokens_t = tokens.T.astype(jnp.int32)
    mesh = plsc.VectorSubcoreMesh(core_axis_name="c", subcore_axis_name="s")
    out = pl.kernel(
        _body,
        out_type=jax.ShapeDtypeStruct((L, EMB // 8, nbb, 8, CHUNK),
                                      jnp.float32),
        mesh=mesh,
        compiler_params=pltpu.CompilerParams(
            use_tc_tiling_on_sc=False, needs_layout_passes=False),
        scratch_types=[
            pltpu.VMEM((L, CHUNK), jnp.int32),
            pltpu.VMEM((NBUF, CHUNK, EMB), jnp.bfloat16),
            pltpu.VMEM((NBUF, EMB // 8, 8, CHUNK + 1), jnp.float32),
        ] + [pltpu.SemaphoreType.DMA] * (2 * NBUF),
    )(tokens_t, embedding_weight.astype(jnp.bfloat16))
    return out.transpose(2, 4, 0, 1, 3).reshape(B, L, EMB)


# speedup vs baseline: 1.2614x; 1.2614x over previous
import jax
import jax.numpy as jnp
from jax import lax
from jax.experimental import pallas as pl
from jax.experimental.pallas import tpu as pltpu
from jax.experimental.pallas import tpu_sc as plsc

EMB = 64
SCALE = 8.0
NC = 2
NS = 16
NW = NC * NS
CHUNK = 128
LANES = 16
NBUF = 4


def _body(tokens_hbm, table_hbm, out_hbm, idx_v, gbuf, tbuf, *sems):
    sem_g = sems[:NBUF]
    sem_w = sems[NBUF:]
    wid = lax.axis_index("s") * NC + lax.axis_index("c")
    nch = tokens_hbm.shape[0]
    pltpu.sync_copy(tokens_hbm.at[:, pl.ds(wid * CHUNK, CHUNK)], idx_v)

    def gather(j, b):
        pltpu.async_copy(table_hbm.at[idx_v.at[j]], gbuf.at[b], sem_g[b])

    def wait_gather(b):
        pltpu.make_async_copy(
            table_hbm.at[idx_v.at[0]], gbuf.at[b], sem_g[b]).wait()

    def writeback(j, b):
        pltpu.async_copy(
            tbuf.at[b, slice(None), slice(None), pl.ds(0, CHUNK)],
            out_hbm.at[j, slice(None), wid], sem_w[b])

    def wait_writeback(b):
        pltpu.make_async_copy(
            tbuf.at[b, slice(None), slice(None), pl.ds(0, CHUNK)],
            out_hbm.at[0, slice(None), wid], sem_w[b]).wait()

    for b in range(NBUF):
        gather(b, b)

    lanes = lax.iota(jnp.int32, LANES)
    eb_c = [(jnp.full((LANES,), c * LANES, jnp.int32) + lanes) >> 3
            for c in range(EMB // LANES)]
    ei_c = [(jnp.full((LANES,), c * LANES, jnp.int32) + lanes) & 7
            for c in range(EMB // LANES)]

    def group(g, carry):
        for b in range(NBUF):
            j = g * NBUF + b
            wait_gather(b)

            @pl.when(j >= NBUF)
            def _():
                wait_writeback(b)

            @plsc.parallel_loop(0, CHUNK, unroll=2)
            def _pass(t):
                t16 = jnp.full((LANES,), 0, jnp.int32) + t
                for c in range(EMB // LANES):
                    v = gbuf[b, t, pl.ds(c * LANES, LANES)] * SCALE
                    plsc.store_scatter(tbuf.at[b], [eb_c[c], ei_c[c], t16], v)

            writeback(j, b)

            @pl.when(j + NBUF < nch)
            def _():
                gather(j + NBUF, b)
        return carry

    lax.fori_loop(0, nch // NBUF, group, None)

    for b in range(NBUF):
        wait_writeback(b)


def kernel(tokens, embedding_weight):
    B, L = tokens.shape
    nbb = B // CHUNK
    assert nbb == NW and L % NBUF == 0, (B, L)
    tokens_t = tokens.T.astype(jnp.int32)
    mesh = plsc.VectorSubcoreMesh(core_axis_name="c", subcore_axis_name="s")
    out = pl.kernel(
        _body,
        out_type=jax.ShapeDtypeStruct((L, EMB // 8, nbb, 8, CHUNK),
                                      jnp.float32),
        mesh=mesh,
        compiler_params=pltpu.CompilerParams(
            use_tc_tiling_on_sc=False, needs_layout_passes=False),
        scratch_types=[
            pltpu.VMEM((L, CHUNK), jnp.int32),
            pltpu.VMEM((NBUF, CHUNK, EMB), jnp.float32),
            pltpu.VMEM((NBUF, EMB // 8, 8, CHUNK + 1), jnp.float32),
        ] + [pltpu.SemaphoreType.DMA] * (2 * NBUF),
    )(tokens_t, embedding_weight)
    return out.transpose(2, 4, 0, 1, 3).reshape(B, L, EMB)
